# baseline (speedup 1.0000x reference)
import itertools

import jax
import jax.numpy as jnp
import numpy as np
from jax import lax
from jax.experimental import pallas as pl
from jax.experimental.pallas import tpu as pltpu
from jax.experimental.pallas import tpu_sc as plsc

N_INPUT_DIMS = 3
N_LEVELS = 16
N_FEATS = 2
LOG2_HASHMAP = 19
BASE_RES = 16
MAX_RES = 512
T = 2**LOG2_HASHMAP
PER_LEVEL_SCALE = float(
    np.exp((np.log(MAX_RES) - np.log(BASE_RES)) / (N_LEVELS - 1))
)
RESOLUTIONS = [
    int(np.floor(BASE_RES * (PER_LEVEL_SCALE**l))) for l in range(N_LEVELS)
]
PRIME1 = int(np.uint32(2654435761).astype(np.int32))
PRIME2 = int(np.uint32(805459861).astype(np.int32))
CORNERS = list(itertools.product([0, 1], repeat=N_INPUT_DIMS))
N_POINTS = 262144

NC = 2
NS = 16
NW = NC * NS
L = 16

NPW = N_POINTS // NW
P = 512
NCHUNK = NPW // P
G = P // L
NFO = N_LEVELS * N_FEATS

N_STAGED = 4


def _staged_sizes():
  offs = []
  off = 0
  for l in range(N_STAGED):
    rows = (RESOLUTIONS[l] + 1) ** N_INPUT_DIMS
    rows = -(-rows // 8) * 8
    offs.append(off)
    off += rows
  return offs, off


CT_OFFS, CT_WORDS = _staged_sizes()


def _wid():
  return lax.axis_index("s") * NC + lax.axis_index("c")


def _start_gather(tabp, idxv, rowsv, sem):
  return [pltpu.async_copy(tabp.at[idxv], rowsv, sem)]


def _unpack2(v):
  b = plsc.bitcast(v, jnp.bfloat16)
  return plsc.unpack(b, format=plsc.PackFormat.INTERLEAVED)


def _corner_geometry(xv0, xv1, xv2, g, res):
  scale = np.float32(res - 1)
  rmax = np.int32(res - 1)
  sl = pl.ds(g * L, L)
  p0c, p1c, frs = [], [], []
  for xv in (xv0, xv1, xv2):
    pos = xv[sl] * scale
    p0 = pos.astype(jnp.int32)
    frs.append(pos - p0.astype(jnp.float32))
    p0c.append(jnp.maximum(jnp.minimum(p0, rmax), 0))
    p1c.append(jnp.minimum(p0 + 1, rmax))
  return p0c, p1c, frs


def _trilerp(frs, vals):
  f0, f1, f2 = frs
  wx = (1.0 - f0, f0)
  wy = (1.0 - f1, f1)
  wz = (1.0 - f2, f2)
  wxy = {(i, j): wx[i] * wy[j] for i in (0, 1) for j in (0, 1)}
  acc0 = jnp.zeros((L,), jnp.float32)
  acc1 = jnp.zeros((L,), jnp.float32)
  for ci, (i, j, k) in enumerate(CORNERS):
    v0, v1 = vals[ci]
    w = wxy[(i, j)] * wz[k]
    acc0 = acc0 + w * v0
    acc1 = acc1 + w * v1
  return acc0, acc1


def _kernel_body(x0_hbm, x1_hbm, x2_hbm, tabp_hbm, out_hbm,
                 xv0, xv1, xv2,
                 fr0a, fr1a, fr2a, fr0b, fr1b, fr2b,
                 idxa, idxb, rowsa, rowsb, outv,
                 ctabp, sema, semb):
  wid = _wid()
  frbuf = ((fr0a, fr1a, fr2a), (fr0b, fr1b, fr2b))
  idxbuf = (idxa, idxb)
  rowsbuf = (rowsa, rowsb)
  sems = (sema, semb)

  for l in range(N_STAGED):
    rows = CT_WORDS - CT_OFFS[l] if l == N_STAGED - 1 else (
        CT_OFFS[l + 1] - CT_OFFS[l])
    src = pl.ds(np.int32(l * T), rows)
    dst = pl.ds(np.int32(CT_OFFS[l]), rows)
    pltpu.sync_copy(tabp_hbm.at[src], ctabp.at[dst])

  def index_level(lvl, b):
    res = RESOLUTIONS[lvl]
    dense = (res + 1) ** N_INPUT_DIMS <= T
    lbase = np.int32(lvl * T)
    fr = frbuf[b]
    idxv = idxbuf[b]

    def index_body(g, _):
      p0c, p1c, frs = _corner_geometry(xv0, xv1, xv2, g, res)
      for d in range(N_INPUT_DIMS):
        fr[d][pl.ds(g * L, L)] = frs[d]
      lane2 = lax.iota(jnp.int32, L) * 2
      if dense:
        s1 = np.int32(res + 1)
        s2 = np.int32((res + 1) * (res + 1))
        ax = (p0c[0], p1c[0])
        by = (p0c[1] * s1, p1c[1] * s1)
        cz = (p0c[2] * s2 + lbase, p1c[2] * s2 + lbase)
        for (i, j, k) in CORNERS:
          pos = lane2 + np.int32((j * 2 + k) * 2 * P + i) + g * (2 * L)
          plsc.store_scatter(idxv, [pos], ax[i] + by[j] + cz[k])
      else:
        mask = np.int32(T - 1)
        hx = (p0c[0], p1c[0])
        hy = (p0c[1] * np.int32(PRIME1), p1c[1] * np.int32(PRIME1))
        hz = (p0c[2] * np.int32(PRIME2), p1c[2] * np.int32(PRIME2))
        for (i, j, k) in CORNERS:
          h = hx[i] ^ hy[j] ^ hz[k]
          pos = lane2 + np.int32((j * 2 + k) * 2 * P + i) + g * (2 * L)
          plsc.store_scatter(idxv, [pos], (h & mask) + lbase)
      return 0

    lax.fori_loop(0, G, index_body, 0)

  def accum_level(lvl, b):
    fr = frbuf[b]
    rowsv = rowsbuf[b]

    def accum_body(g, _):
      sl = pl.ds(g * L, L)
      frs = (fr[0][sl], fr[1][sl], fr[2][sl])
      lane2 = lax.iota(jnp.int32, L) * 2
      vals = []
      for (i, j, k) in CORNERS:
        pos = lane2 + np.int32((j * 2 + k) * 2 * P + i) + g * (2 * L)
        vals.append(_unpack2(plsc.load_gather(rowsv, [pos])))
      acc0, acc1 = _trilerp(frs, vals)
      pt0 = lax.iota(jnp.int32, L) * NFO + (g * (L * NFO) + 2 * lvl)
      plsc.store_scatter(outv, [pt0], acc0)
      plsc.store_scatter(outv, [pt0 + 1], acc1)
      return 0

    lax.fori_loop(0, G, accum_body, 0)

  def staged_level(lvl):
    res = RESOLUTIONS[lvl]
    off = np.int32(CT_OFFS[lvl])
    s1 = np.int32(res + 1)
    s2 = np.int32((res + 1) * (res + 1))

    def body(g, _):
      p0c, p1c, frs = _corner_geometry(xv0, xv1, xv2, g, res)
      ax = (p0c[0], p1c[0])
      by = (p0c[1] * s1, p1c[1] * s1)
      cz = (p0c[2] * s2 + off, p1c[2] * s2 + off)
      vals = []
      for (i, j, k) in CORNERS:
        idx = ax[i] + by[j] + cz[k]
        vals.append(_unpack2(plsc.load_gather(ctabp, [idx])))
      acc0, acc1 = _trilerp(frs, vals)
      pt0 = lax.iota(jnp.int32, L) * NFO + (g * (L * NFO) + 2 * lvl)
      plsc.store_scatter(outv, [pt0], acc0)
      plsc.store_scatter(outv, [pt0 + 1], acc1)
      return 0

    lax.fori_loop(0, G, body, 0)

  def load_x(c):
    base = (wid * NPW + c * P).astype(jnp.int32)
    pltpu.sync_copy(x0_hbm.at[pl.ds(base, P)], xv0)
    pltpu.sync_copy(x1_hbm.at[pl.ds(base, P)], xv1)
    pltpu.sync_copy(x2_hbm.at[pl.ds(base, P)], xv2)

  load_x(0)
  index_level(N_STAGED, 0)
  desc = {0: _start_gather(tabp_hbm, idxbuf[0], rowsbuf[0], sems[0])}

  def chunk_body(c, _):
    base = (wid * NPW + c * P).astype(jnp.int32)
    for lvl in range(N_STAGED):
      staged_level(lvl)
    cnext = jnp.minimum(c + 1, NCHUNK - 1)
    for lvl in range(N_STAGED, N_LEVELS):
      cb = (lvl - N_STAGED) % 2
      nb = 1 - cb
      if lvl + 1 < N_LEVELS:
        index_level(lvl + 1, nb)
        desc[nb] = _start_gather(tabp_hbm, idxbuf[nb], rowsbuf[nb], sems[nb])
        if lvl + 1 == N_LEVELS - 1:
          load_x(cnext)
      else:
        index_level(N_STAGED, nb)
        desc[nb] = _start_gather(tabp_hbm, idxbuf[nb], rowsbuf[nb], sems[nb])
      for d in desc[cb]:
        d.wait()
      accum_level(lvl, cb)

    pltpu.sync_copy(outv, out_hbm.at[pl.ds(base * NFO, P * NFO)])
    return 0

  lax.fori_loop(0, NCHUNK, chunk_body, 0)
  for d in desc[(N_LEVELS - N_STAGED) % 2]:
    d.wait()


def kernel(x, table):
  x0 = x[:, 0]
  x1 = x[:, 1]
  x2 = x[:, 2]
  tabp = jax.lax.bitcast_convert_type(
      table.astype(jnp.bfloat16), jnp.float32).reshape(N_LEVELS * T)

  mesh = plsc.VectorSubcoreMesh(core_axis_name="c", subcore_axis_name="s",
                                num_cores=NC, num_subcores=NS)
  f = pl.kernel(
      _kernel_body,
      out_type=jax.ShapeDtypeStruct((N_POINTS * NFO,), jnp.float32),
      mesh=mesh,
      compiler_params=pltpu.CompilerParams(use_tc_tiling_on_sc=False,
                                           needs_layout_passes=False),
      scratch_types=(
          [pltpu.VMEM((P,), jnp.float32)] * 9
          + [pltpu.VMEM((8 * P,), jnp.int32)] * 2
          + [pltpu.VMEM((8 * P,), jnp.float32)] * 2
          + [pltpu.VMEM((P * NFO,), jnp.float32)]
          + [pltpu.VMEM((CT_WORDS,), jnp.float32)]
          + [pltpu.SemaphoreType.DMA] * 2
      ),
  )
  return f(x0, x1, x2, tabp).reshape(N_POINTS, NFO)


# speedup vs baseline: 1.0012x; 1.0012x over previous
import itertools

import jax
import jax.numpy as jnp
import numpy as np
from jax import lax
from jax.experimental import pallas as pl
from jax.experimental.pallas import tpu as pltpu
from jax.experimental.pallas import tpu_sc as plsc

N_INPUT_DIMS = 3
N_LEVELS = 16
N_FEATS = 2
LOG2_HASHMAP = 19
BASE_RES = 16
MAX_RES = 512
T = 2**LOG2_HASHMAP
PER_LEVEL_SCALE = float(
    np.exp((np.log(MAX_RES) - np.log(BASE_RES)) / (N_LEVELS - 1))
)
RESOLUTIONS = [
    int(np.floor(BASE_RES * (PER_LEVEL_SCALE**l))) for l in range(N_LEVELS)
]
PRIME1 = int(np.uint32(2654435761).astype(np.int32))
PRIME2 = int(np.uint32(805459861).astype(np.int32))
CORNERS = list(itertools.product([0, 1], repeat=N_INPUT_DIMS))
N_POINTS = 262144

NC = 2
NS = 16
NW = NC * NS
L = 16

NPW = N_POINTS // NW
P = 512
NCHUNK = NPW // P
G = P // L
NFO = N_LEVELS * N_FEATS

N_STAGED = 4


def _staged_sizes():
  offs = []
  off = 0
  for l in range(N_STAGED):
    rows = (RESOLUTIONS[l] + 1) ** N_INPUT_DIMS
    rows = -(-rows // 8) * 8
    offs.append(off)
    off += rows
  return offs, off


CT_OFFS, CT_WORDS = _staged_sizes()


def _wid():
  return lax.axis_index("s") * NC + lax.axis_index("c")


def _start_gather(tabp, idxv, rowsv, sem):
  return [pltpu.async_copy(tabp.at[idxv], rowsv, sem)]


def _unpack2(v):
  b = plsc.bitcast(v, jnp.bfloat16)
  return plsc.unpack(b, format=plsc.PackFormat.INTERLEAVED)


def _corner_geometry(xv0, xv1, xv2, g, res):
  scale = np.float32(res - 1)
  rmax = np.int32(res - 1)
  sl = pl.ds(g * L, L)
  p0c, p1c, frs = [], [], []
  for xv in (xv0, xv1, xv2):
    pos = xv[sl] * scale
    p0 = pos.astype(jnp.int32)
    frs.append(pos - p0.astype(jnp.float32))
    p0c.append(jnp.maximum(jnp.minimum(p0, rmax), 0))
    p1c.append(jnp.minimum(p0 + 1, rmax))
  return p0c, p1c, frs


def _trilerp(frs, vals):
  f0, f1, f2 = frs
  wx = (1.0 - f0, f0)
  wy = (1.0 - f1, f1)
  wz = (1.0 - f2, f2)
  wxy = {(i, j): wx[i] * wy[j] for i in (0, 1) for j in (0, 1)}
  acc0 = jnp.zeros((L,), jnp.float32)
  acc1 = jnp.zeros((L,), jnp.float32)
  for ci, (i, j, k) in enumerate(CORNERS):
    v0, v1 = vals[ci]
    w = wxy[(i, j)] * wz[k]
    acc0 = acc0 + w * v0
    acc1 = acc1 + w * v1
  return acc0, acc1


def _kernel_body(x0_hbm, x1_hbm, x2_hbm, tabp_hbm, out_hbm,
                 xv0, xv1, xv2,
                 fr0a, fr1a, fr2a, fr0b, fr1b, fr2b,
                 idxa, idxb, rowsa, rowsb, outv,
                 ctabp, sema, semb):
  wid = _wid()
  frbuf = ((fr0a, fr1a, fr2a), (fr0b, fr1b, fr2b))
  idxbuf = (idxa, idxb)
  rowsbuf = (rowsa, rowsb)
  sems = (sema, semb)

  for l in range(N_STAGED):
    rows = CT_WORDS - CT_OFFS[l] if l == N_STAGED - 1 else (
        CT_OFFS[l + 1] - CT_OFFS[l])
    src = pl.ds(np.int32(l * T), rows)
    dst = pl.ds(np.int32(CT_OFFS[l]), rows)
    pltpu.sync_copy(tabp_hbm.at[src], ctabp.at[dst])

  def index_level(lvl, b):
    res = RESOLUTIONS[lvl]
    dense = (res + 1) ** N_INPUT_DIMS <= T
    lbase = np.int32(lvl * T)
    fr = frbuf[b]
    idxv = idxbuf[b]

    def index_body(g, _):
      p0c, p1c, frs = _corner_geometry(xv0, xv1, xv2, g, res)
      for d in range(N_INPUT_DIMS):
        fr[d][pl.ds(g * L, L)] = frs[d]
      if dense:
        s1 = np.int32(res + 1)
        s2 = np.int32((res + 1) * (res + 1))
        ax = (p0c[0], p1c[0])
        by = (p0c[1] * s1, p1c[1] * s1)
        cz = (p0c[2] * s2 + lbase, p1c[2] * s2 + lbase)
        for ci, (i, j, k) in enumerate(CORNERS):
          idxv[pl.ds(np.int32(ci * P) + g * L, L)] = ax[i] + by[j] + cz[k]
      else:
        mask = np.int32(T - 1)
        hx = (p0c[0], p1c[0])
        hy = (p0c[1] * np.int32(PRIME1), p1c[1] * np.int32(PRIME1))
        hz = (p0c[2] * np.int32(PRIME2), p1c[2] * np.int32(PRIME2))
        for ci, (i, j, k) in enumerate(CORNERS):
          h = hx[i] ^ hy[j] ^ hz[k]
          idxv[pl.ds(np.int32(ci * P) + g * L, L)] = (h & mask) + lbase
      return 0

    lax.fori_loop(0, G, index_body, 0)

  def accum_level(lvl, b):
    fr = frbuf[b]
    rowsv = rowsbuf[b]

    def accum_body(g, _):
      sl = pl.ds(g * L, L)
      frs = (fr[0][sl], fr[1][sl], fr[2][sl])
      vals = []
      for ci in range(8):
        rsl = pl.ds(np.int32(ci * P) + g * L, L)
        vals.append(_unpack2(rowsv[rsl]))
      acc0, acc1 = _trilerp(frs, vals)
      pt0 = lax.iota(jnp.int32, L) * NFO + (g * (L * NFO) + 2 * lvl)
      plsc.store_scatter(outv, [pt0], acc0)
      plsc.store_scatter(outv, [pt0 + 1], acc1)
      return 0

    lax.fori_loop(0, G, accum_body, 0)

  def staged_level(lvl):
    res = RESOLUTIONS[lvl]
    off = np.int32(CT_OFFS[lvl])
    s1 = np.int32(res + 1)
    s2 = np.int32((res + 1) * (res + 1))

    def body(g, _):
      p0c, p1c, frs = _corner_geometry(xv0, xv1, xv2, g, res)
      ax = (p0c[0], p1c[0])
      by = (p0c[1] * s1, p1c[1] * s1)
      cz = (p0c[2] * s2 + off, p1c[2] * s2 + off)
      vals = []
      for (i, j, k) in CORNERS:
        idx = ax[i] + by[j] + cz[k]
        vals.append(_unpack2(plsc.load_gather(ctabp, [idx])))
      acc0, acc1 = _trilerp(frs, vals)
      pt0 = lax.iota(jnp.int32, L) * NFO + (g * (L * NFO) + 2 * lvl)
      plsc.store_scatter(outv, [pt0], acc0)
      plsc.store_scatter(outv, [pt0 + 1], acc1)
      return 0

    lax.fori_loop(0, G, body, 0)

  def load_x(c):
    base = (wid * NPW + c * P).astype(jnp.int32)
    pltpu.sync_copy(x0_hbm.at[pl.ds(base, P)], xv0)
    pltpu.sync_copy(x1_hbm.at[pl.ds(base, P)], xv1)
    pltpu.sync_copy(x2_hbm.at[pl.ds(base, P)], xv2)

  load_x(0)
  index_level(N_STAGED, 0)
  desc = {0: _start_gather(tabp_hbm, idxbuf[0], rowsbuf[0], sems[0])}

  def chunk_body(c, _):
    base = (wid * NPW + c * P).astype(jnp.int32)
    for lvl in range(N_STAGED):
      staged_level(lvl)
    cnext = jnp.minimum(c + 1, NCHUNK - 1)
    for lvl in range(N_STAGED, N_LEVELS):
      cb = (lvl - N_STAGED) % 2
      nb = 1 - cb
      if lvl + 1 < N_LEVELS:
        index_level(lvl + 1, nb)
        desc[nb] = _start_gather(tabp_hbm, idxbuf[nb], rowsbuf[nb], sems[nb])
        if lvl + 1 == N_LEVELS - 1:
          load_x(cnext)
      else:
        index_level(N_STAGED, nb)
        desc[nb] = _start_gather(tabp_hbm, idxbuf[nb], rowsbuf[nb], sems[nb])
      for d in desc[cb]:
        d.wait()
      accum_level(lvl, cb)

    pltpu.sync_copy(outv, out_hbm.at[pl.ds(base * NFO, P * NFO)])
    return 0

  lax.fori_loop(0, NCHUNK, chunk_body, 0)
  for d in desc[(N_LEVELS - N_STAGED) % 2]:
    d.wait()


def kernel(x, table):
  x0 = x[:, 0]
  x1 = x[:, 1]
  x2 = x[:, 2]
  tabp = jax.lax.bitcast_convert_type(
      table.astype(jnp.bfloat16), jnp.float32).reshape(N_LEVELS * T)

  mesh = plsc.VectorSubcoreMesh(core_axis_name="c", subcore_axis_name="s",
                                num_cores=NC, num_subcores=NS)
  f = pl.kernel(
      _kernel_body,
      out_type=jax.ShapeDtypeStruct((N_POINTS * NFO,), jnp.float32),
      mesh=mesh,
      compiler_params=pltpu.CompilerParams(use_tc_tiling_on_sc=False,
                                           needs_layout_passes=False),
      scratch_types=(
          [pltpu.VMEM((P,), jnp.float32)] * 9
          + [pltpu.VMEM((8 * P,), jnp.int32)] * 2
          + [pltpu.VMEM((8 * P,), jnp.float32)] * 2
          + [pltpu.VMEM((P * NFO,), jnp.float32)]
          + [pltpu.VMEM((CT_WORDS,), jnp.float32)]
          + [pltpu.SemaphoreType.DMA] * 2
      ),
  )
  return f(x0, x1, x2, tabp).reshape(N_POINTS, NFO)
